# baseline (speedup 1.0000x reference)
import functools

import jax
import jax.numpy as jnp
from jax import lax
from jax.experimental import pallas as pl
from jax.experimental.pallas import tpu as pltpu, tpu_sc as plsc

NC = 2
NS = 16
L = 16
NW = NC * NS


def _gather_add_kernel(B, D, V):
    b_per_w = B // NW
    CH = 32
    n_chunks = b_per_w // CH
    vecs_per_row = D // L

    mesh = plsc.VectorSubcoreMesh(core_axis_name="c", subcore_axis_name="s")

    @functools.partial(
        pl.kernel,
        out_type=jax.ShapeDtypeStruct((B, D), jnp.float32),
        mesh=mesh,
        scratch_types=[
            pltpu.VMEM((CH,), jnp.int32),
            pltpu.VMEM((CH, D), jnp.float32),
            pltpu.VMEM((CH, D), jnp.float32),
            pltpu.SemaphoreType.DMA,
        ],
    )
    def body(x_hbm, emb_hbm, idx_hbm, out_hbm, idx_v, rows_v, x_v, sem):
        wid = lax.axis_index("s") * NC + lax.axis_index("c")
        base = wid * b_per_w

        def chunk(c, _):
            cb = base + c * CH
            pltpu.sync_copy(idx_hbm.at[pl.ds(cb, CH)], idx_v)
            gather = pltpu.async_copy(emb_hbm.at[idx_v], rows_v, sem)
            pltpu.sync_copy(x_hbm.at[pl.ds(cb, CH), :], x_v)
            gather.wait()

            def row(r, _):
                for j in range(vecs_per_row):
                    sl = pl.ds(j * L, L)
                    plsc.addupdate(rows_v.at[r, sl], x_v[r, sl])
                return 0

            lax.fori_loop(0, CH, row, 0)
            pltpu.sync_copy(rows_v, out_hbm.at[pl.ds(cb, CH), :])
            return 0

        lax.fori_loop(0, n_chunks, chunk, 0)

    return body


@jax.jit
def _run(x_tc, embeddings_tc, times_t):
    B, D = x_tc.shape
    V = embeddings_tc.shape[0]
    fn = _gather_add_kernel(B, D, V)
    return fn(x_tc, embeddings_tc, times_t.astype(jnp.int32))


def kernel(x_tc, embeddings_tc, offset, times_t):
    if times_t is None:
        times_t = offset + jnp.arange(x_tc.shape[0], dtype=jnp.int32)
    return _run(x_tc, embeddings_tc, times_t)
